# baseline (speedup 1.0000x reference)
import functools
import jax
import jax.numpy as jnp
from jax import lax
from jax.experimental import pallas as pl
from jax.experimental.pallas import tpu as pltpu
from jax.experimental.pallas import tpu_sc as plsc

NC = 2
NS = 16
NW = NC * NS
CH = 16
D = 1024
NV = D // 16


def kernel(inputs, weight):
    batch, seq_len, dim = inputs.shape
    per_w = seq_len // NW
    n_chunks = per_w // CH

    mesh = plsc.VectorSubcoreMesh(core_axis_name="c", subcore_axis_name="s")

    @functools.partial(
        pl.kernel,
        mesh=mesh,
        out_type=jax.ShapeDtypeStruct((batch, seq_len, dim), jnp.float32),
        scratch_types=[
            pltpu.VMEM((CH, D), jnp.float32),
            pltpu.VMEM((batch * CH, D), jnp.float32),
        ],
    )
    def sc_k(x_hbm, w_hbm, out_hbm, wbuf, xbuf):
        cid = lax.axis_index("c")
        sid = lax.axis_index("s")
        wid = sid * NC + cid
        for t in range(n_chunks):
            s0 = wid * per_w + t * CH
            pltpu.sync_copy(w_hbm.at[pl.ds(s0, CH)], wbuf)
            for b in range(batch):
                pltpu.sync_copy(
                    x_hbm.at[b, pl.ds(s0, CH)], xbuf.at[pl.ds(b * CH, CH)]
                )

            def body(i, _):
                r = i // NV
                c = (i - r * NV) * 16
                wv = wbuf[r, pl.ds(c, 16)]
                for b in range(batch):
                    xbuf[b * CH + r, pl.ds(c, 16)] = (
                        xbuf[b * CH + r, pl.ds(c, 16)] + wv
                    )
                return 0

            lax.fori_loop(0, CH * NV, body, 0)
            for b in range(batch):
                pltpu.sync_copy(
                    xbuf.at[pl.ds(b * CH, CH)], out_hbm.at[b, pl.ds(s0, CH)]
                )

    return sc_k(inputs, weight)


# speedup vs baseline: 1.7304x; 1.7304x over previous
import functools
import jax
import jax.numpy as jnp
from jax import lax
from jax.experimental import pallas as pl
from jax.experimental.pallas import tpu as pltpu
from jax.experimental.pallas import tpu_sc as plsc

NC = 2
NS = 16
NW = NC * NS
CH = 8
D = 1024
NV = D // 16


def kernel(inputs, weight):
    batch, seq_len, dim = inputs.shape
    per_w = seq_len // NW
    n_chunks = per_w // CH

    mesh = plsc.VectorSubcoreMesh(core_axis_name="c", subcore_axis_name="s")

    @functools.partial(
        pl.kernel,
        mesh=mesh,
        out_type=jax.ShapeDtypeStruct((batch, seq_len, dim), jnp.float32),
        scratch_types=[
            pltpu.VMEM((2, CH, D), jnp.float32),
            pltpu.VMEM((2, batch * CH, D), jnp.float32),
            pltpu.SemaphoreType.DMA,
            pltpu.SemaphoreType.DMA,
            pltpu.SemaphoreType.DMA,
            pltpu.SemaphoreType.DMA,
        ],
    )
    def sc_k(x_hbm, w_hbm, out_hbm, wbuf, xbuf, semw, semx, semo0, semo1):
        cid = lax.axis_index("c")
        sid = lax.axis_index("s")
        wid = sid * NC + cid
        semo = (semo0, semo1)

        def start_loads(t, p):
            s0 = wid * per_w + t * CH
            hw = pltpu.async_copy(w_hbm.at[pl.ds(s0, CH)], wbuf.at[p], semw)
            hx = []
            for b in range(batch):
                hx.append(
                    pltpu.async_copy(
                        x_hbm.at[b, pl.ds(s0, CH)],
                        xbuf.at[p, pl.ds(b * CH, CH)],
                        semx,
                    )
                )
            return hw, hx

        def start_stores(t, p):
            s0 = wid * per_w + t * CH
            hs = []
            for b in range(batch):
                hs.append(
                    pltpu.async_copy(
                        xbuf.at[p, pl.ds(b * CH, CH)],
                        out_hbm.at[b, pl.ds(s0, CH)],
                        semo[p],
                    )
                )
            return hs

        loads = start_loads(0, 0)
        stores = [None, None]
        for t in range(n_chunks):
            p = t % 2
            if t + 1 < n_chunks:
                next_loads = start_loads(t + 1, 1 - p)
            hw, hx = loads
            hw.wait()
            for h in hx:
                h.wait()

            @plsc.parallel_loop(0, CH * NV, 1, unroll=8)
            def body(i):
                r = i // NV
                c = (i - r * NV) * 16
                wv = wbuf[p, r, pl.ds(c, 16)]
                for b in range(batch):
                    xbuf[p, b * CH + r, pl.ds(c, 16)] = (
                        xbuf[p, b * CH + r, pl.ds(c, 16)] + wv
                    )
            if stores[p] is not None:
                for h in stores[p]:
                    h.wait()
            stores[p] = start_stores(t, p)
            if t + 1 < n_chunks:
                loads = next_loads
        for hs in stores:
            if hs is not None:
                for h in hs:
                    h.wait()

    return sc_k(inputs, weight)


# speedup vs baseline: 1.7465x; 1.0093x over previous
import functools
import jax
import jax.numpy as jnp
from jax import lax
from jax.experimental import pallas as pl
from jax.experimental.pallas import tpu as pltpu
from jax.experimental.pallas import tpu_sc as plsc

NC = 2
NS = 16
NW = NC * NS
CH = 8
D = 1024
NV = D // 16
NBUF = 3


def kernel(inputs, weight):
    batch, seq_len, dim = inputs.shape
    per_w = seq_len // NW
    n_chunks = per_w // CH

    mesh = plsc.VectorSubcoreMesh(core_axis_name="c", subcore_axis_name="s")

    @functools.partial(
        pl.kernel,
        mesh=mesh,
        out_type=jax.ShapeDtypeStruct((batch, seq_len, dim), jnp.float32),
        scratch_types=[
            pltpu.VMEM((NBUF, CH, D), jnp.float32),
            pltpu.VMEM((NBUF, batch * CH, D), jnp.float32),
            pltpu.SemaphoreType.DMA,
            pltpu.SemaphoreType.DMA,
            pltpu.SemaphoreType.DMA,
            pltpu.SemaphoreType.DMA,
            pltpu.SemaphoreType.DMA,
        ],
    )
    def sc_k(x_hbm, w_hbm, out_hbm, wbuf, xbuf, semw, semx, semo0, semo1, semo2):
        cid = lax.axis_index("c")
        sid = lax.axis_index("s")
        wid = sid * NC + cid
        semo = (semo0, semo1, semo2)

        def start_loads(t, p):
            s0 = wid * per_w + t * CH
            hw = pltpu.async_copy(w_hbm.at[pl.ds(s0, CH)], wbuf.at[p], semw)
            hx = []
            for b in range(batch):
                hx.append(
                    pltpu.async_copy(
                        x_hbm.at[b, pl.ds(s0, CH)],
                        xbuf.at[p, pl.ds(b * CH, CH)],
                        semx,
                    )
                )
            return hw, hx

        def start_stores(t, p):
            s0 = wid * per_w + t * CH
            hs = []
            for b in range(batch):
                hs.append(
                    pltpu.async_copy(
                        xbuf.at[p, pl.ds(b * CH, CH)],
                        out_hbm.at[b, pl.ds(s0, CH)],
                        semo[p],
                    )
                )
            return hs

        loads = [None] * NBUF
        stores = [None] * NBUF
        loads[0] = start_loads(0, 0)
        loads[1] = start_loads(1, 1)
        for t in range(n_chunks):
            p = t % NBUF
            if t + 2 < n_chunks:
                q = (t + 2) % NBUF
                if stores[q] is not None:
                    for h in stores[q]:
                        h.wait()
                    stores[q] = None
                loads[q] = start_loads(t + 2, q)
            hw, hx = loads[p]
            hw.wait()
            for h in hx:
                h.wait()

            @plsc.parallel_loop(0, CH * NV, 1, unroll=8)
            def body(i):
                r = i // NV
                c = (i - r * NV) * 16
                wv = wbuf[p, r, pl.ds(c, 16)]
                for b in range(batch):
                    xbuf[p, b * CH + r, pl.ds(c, 16)] = (
                        xbuf[p, b * CH + r, pl.ds(c, 16)] + wv
                    )

            stores[p] = start_stores(t, p)
        for hs in stores:
            if hs is not None:
                for h in hs:
                    h.wait()

    return sc_k(inputs, weight)


# speedup vs baseline: 2.7516x; 1.5755x over previous
import jax
import jax.numpy as jnp
from jax.experimental import pallas as pl

BLOCK_S = 512


def _add_kernel(x_ref, w_ref, o_ref):
    o_ref[...] = x_ref[...] + w_ref[...]


def kernel(inputs, weight):
    batch, seq_len, dim = inputs.shape
    w = weight[:seq_len]
    grid = (seq_len // BLOCK_S,)
    return pl.pallas_call(
        _add_kernel,
        grid=grid,
        in_specs=[
            pl.BlockSpec((batch, BLOCK_S, dim), lambda s: (0, s, 0)),
            pl.BlockSpec((BLOCK_S, dim), lambda s: (s, 0)),
        ],
        out_specs=pl.BlockSpec((batch, BLOCK_S, dim), lambda s: (0, s, 0)),
        out_shape=jax.ShapeDtypeStruct(inputs.shape, inputs.dtype),
    )(inputs, w)
